# baseline (speedup 1.0000x reference)
import functools

import jax
import jax.numpy as jnp
from jax import lax
from jax.experimental import pallas as pl
from jax.experimental.pallas import tpu as pltpu
from jax.experimental.pallas import tpu_sc as plsc

BATCH = 16384
POOL = 20
EMBED_DIM = 64
NB = 4096
NC, NS, L = 2, 16, 16
NW = NC * NS
NIDX = BATCH * POOL
PER_W = NIDX // NW


IDX_BLK = 1024
GRID = BATCH // IDX_BLK
PADP = 32
SENT = 4000
OROWS_BLK = IDX_BLK * PADP // 128
NIDXP = BATCH * PADP


HI = 64


def _repack_block(xi):
    hi = (xi // HI).astype(jnp.bfloat16)
    lo = (xi % HI).astype(jnp.bfloat16)
    rhs = jnp.concatenate([hi, lo], axis=1)
    r_out = lax.broadcasted_iota(jnp.int32, (OROWS_BLK, IDX_BLK), 0)
    r_in = lax.broadcasted_iota(jnp.int32, (OROWS_BLK, IDX_BLK), 1)
    parts = []
    for q in range(4):
        sel = (r_in == 4 * r_out + q).astype(jnp.bfloat16)
        p = jnp.dot(sel, rhs, preferred_element_type=jnp.float32)
        parts.append((p[:, :PADP] * HI + p[:, PADP:]).astype(jnp.int32))
    return jnp.concatenate(parts, axis=1)


def _tc_prep_body(idx0_ref, idx1_ref, t0_ref, t1_ref, o0_ref, o1_ref, rs_ref):
    pad = jnp.full((IDX_BLK, PADP - POOL), SENT, jnp.int32)
    for iref, oref in ((idx0_ref, o0_ref), (idx1_ref, o1_ref)):
        xi = jnp.concatenate([iref[...], pad], axis=1)
        oref[...] = _repack_block(xi)

    @pl.when(pl.program_id(0) == 0)
    def _():
        keep = (lax.broadcasted_iota(jnp.int32, (NB,), 0) < SENT).astype(jnp.float32)
        rs_ref[0, :] = jnp.sum(t0_ref[...], axis=1) * keep
        rs_ref[1, :] = jnp.sum(t1_ref[...], axis=1) * keep


_tc_prep = pl.pallas_call(
    _tc_prep_body,
    grid=(GRID,),
    out_shape=(
        jax.ShapeDtypeStruct((NIDXP // 128, 128), jnp.int32),
        jax.ShapeDtypeStruct((NIDXP // 128, 128), jnp.int32),
        jax.ShapeDtypeStruct((2, NB), jnp.float32),
    ),
    in_specs=[
        pl.BlockSpec((IDX_BLK, POOL), lambda i: (i, 0)),
        pl.BlockSpec((IDX_BLK, POOL), lambda i: (i, 0)),
        pl.BlockSpec((NB, EMBED_DIM), lambda i: (0, 0)),
        pl.BlockSpec((NB, EMBED_DIM), lambda i: (0, 0)),
    ],
    out_specs=(
        pl.BlockSpec((OROWS_BLK, 128), lambda i: (i, 0)),
        pl.BlockSpec((OROWS_BLK, 128), lambda i: (i, 0)),
        pl.BlockSpec((2, NB), lambda i: (0, 0)),
    ),
)

_mesh = plsc.VectorSubcoreMesh(
    core_axis_name="c", subcore_axis_name="s", num_cores=NC, num_subcores=NS
)


ROWS = NIDXP // 128
ROWS_W = ROWS // NW

_SC_SCRATCH = [
    pltpu.VMEM((NB,), jnp.float32),
    pltpu.VMEM((NB,), jnp.float32),
    pltpu.VMEM((ROWS_W, 128), jnp.int32),
    pltpu.VMEM((ROWS_W, 128), jnp.int32),
    pltpu.VMEM((L,), jnp.float32),
]


def _sc_gather_sum_body(rs_hbm, idx0_hbm, idx1_hbm, out_hbm,
                        lut0, lut1, idx0_v, idx1_v, acc_v):
    wid = lax.axis_index("s") * NC + lax.axis_index("c")
    base = wid * ROWS_W
    pltpu.sync_copy(rs_hbm.at[0], lut0)
    pltpu.sync_copy(rs_hbm.at[1], lut1)
    pltpu.sync_copy(idx0_hbm.at[pl.ds(base, ROWS_W)], idx0_v)
    pltpu.sync_copy(idx1_hbm.at[pl.ds(base, ROWS_W)], idx1_v)

    def body(r, acc):
        for c in range(128 // L):
            iv0 = idx0_v[r, pl.ds(c * L, L)]
            iv1 = idx1_v[r, pl.ds(c * L, L)]
            acc = acc + plsc.load_gather(lut0, [iv0]) + plsc.load_gather(lut1, [iv1])
        return acc

    acc = lax.fori_loop(0, ROWS_W, body, jnp.zeros((L,), jnp.float32))
    acc_v[...] = acc
    pltpu.sync_copy(acc_v, out_hbm.at[wid])


_sc_gather_sum = pl.kernel(
    _sc_gather_sum_body,
    out_type=jax.ShapeDtypeStruct((NW, L), jnp.float32),
    mesh=_mesh,
    scratch_types=_SC_SCRATCH,
    compiler_params=pltpu.CompilerParams(needs_layout_passes=False),
)


def kernel(indices_0, indices_1, table_0, table_1):
    i0f, i1f, rs = _tc_prep(indices_0, indices_1, table_0, table_1)
    partials = _sc_gather_sum(rs, i0f, i1f)
    return jnp.sum(partials) / jnp.float32(BATCH * 2 * EMBED_DIM)


# speedup vs baseline: 5.0729x; 5.0729x over previous
import jax
import jax.numpy as jnp
from jax import lax
from jax.experimental import pallas as pl
from jax.experimental.pallas import tpu as pltpu
from jax.experimental.pallas import tpu_sc as plsc

BATCH = 16384
POOL = 20
EMBED_DIM = 64
NB = 4096
NC, NS, L = 2, 16, 16
NW = NC * NS
COLS_W = BATCH // NW
ITERS = POOL * COLS_W // L


def _rowsum_body(t0_ref, t1_ref, rs_ref):
    rs_ref[0, :] = jnp.sum(t0_ref[...], axis=0)
    rs_ref[1, :] = jnp.sum(t1_ref[...], axis=0)


_rowsum = pl.pallas_call(
    _rowsum_body,
    grid=(1,),
    out_shape=jax.ShapeDtypeStruct((2, NB), jnp.float32),
    in_specs=[
        pl.BlockSpec((EMBED_DIM, NB), lambda i: (0, 0)),
        pl.BlockSpec((EMBED_DIM, NB), lambda i: (0, 0)),
    ],
    out_specs=pl.BlockSpec((2, NB), lambda i: (0, 0)),
)

_mesh = plsc.VectorSubcoreMesh(
    core_axis_name="c", subcore_axis_name="s", num_cores=NC, num_subcores=NS
)

_SC_SCRATCH = [
    pltpu.VMEM((NB,), jnp.float32),
    pltpu.VMEM((NB,), jnp.float32),
    pltpu.VMEM((POOL, COLS_W), jnp.int32),
    pltpu.VMEM((POOL, COLS_W), jnp.int32),
    pltpu.VMEM((L,), jnp.float32),
]


def _sc_gather_sum_body(rs_hbm, it0_hbm, it1_hbm, out_hbm,
                        lut0, lut1, idx0_v, idx1_v, acc_v):
    wid = lax.axis_index("s") * NC + lax.axis_index("c")
    base = wid * COLS_W
    pltpu.sync_copy(rs_hbm.at[0], lut0)
    pltpu.sync_copy(rs_hbm.at[1], lut1)
    pltpu.sync_copy(it0_hbm.at[:, pl.ds(base, COLS_W)], idx0_v)
    pltpu.sync_copy(it1_hbm.at[:, pl.ds(base, COLS_W)], idx1_v)

    npc = COLS_W // L

    def body(i, acc):
        r = i // npc
        c = lax.rem(i, npc)
        iv0 = idx0_v[r, pl.ds(c * L, L)]
        iv1 = idx1_v[r, pl.ds(c * L, L)]
        return acc + plsc.load_gather(lut0, [iv0]) + plsc.load_gather(lut1, [iv1])

    acc = lax.fori_loop(0, ITERS, body, jnp.zeros((L,), jnp.float32))
    acc_v[...] = acc
    pltpu.sync_copy(acc_v, out_hbm.at[wid])


_sc_gather_sum = pl.kernel(
    _sc_gather_sum_body,
    out_type=jax.ShapeDtypeStruct((NW, L), jnp.float32),
    mesh=_mesh,
    scratch_types=_SC_SCRATCH,
    compiler_params=pltpu.CompilerParams(needs_layout_passes=False),
)


def kernel(indices_0, indices_1, table_0, table_1):
    rs = _rowsum(table_0.T, table_1.T)
    partials = _sc_gather_sum(rs, indices_0.T, indices_1.T)
    return jnp.sum(partials) / jnp.float32(BATCH * 2 * EMBED_DIM)


# speedup vs baseline: 5.7799x; 1.1394x over previous
import jax
import jax.numpy as jnp
from jax import lax
from jax.experimental import pallas as pl
from jax.experimental.pallas import tpu as pltpu
from jax.experimental.pallas import tpu_sc as plsc

BATCH = 16384
POOL = 20
EMBED_DIM = 64
NB = 4096
NC, NS, L = 2, 16, 16
NW = NC * NS
COLS_W = BATCH // NW
ITERS = POOL * COLS_W // L


def _rowsum_body(t0_ref, t1_ref, rs_ref):
    rs_ref[0, :] = jnp.sum(t0_ref[...], axis=0)
    rs_ref[1, :] = jnp.sum(t1_ref[...], axis=0)


_rowsum = pl.pallas_call(
    _rowsum_body,
    grid=(1,),
    out_shape=jax.ShapeDtypeStruct((2, NB), jnp.float32),
    in_specs=[
        pl.BlockSpec((EMBED_DIM, NB), lambda i: (0, 0)),
        pl.BlockSpec((EMBED_DIM, NB), lambda i: (0, 0)),
    ],
    out_specs=pl.BlockSpec((2, NB), lambda i: (0, 0)),
)

_mesh = plsc.VectorSubcoreMesh(
    core_axis_name="c", subcore_axis_name="s", num_cores=NC, num_subcores=NS
)

_SC_SCRATCH = [
    pltpu.VMEM((NB,), jnp.float32),
    pltpu.VMEM((NB,), jnp.float32),
    pltpu.VMEM((POOL, COLS_W), jnp.int32),
    pltpu.VMEM((POOL, COLS_W), jnp.int32),
    pltpu.VMEM((L,), jnp.float32),
    pltpu.SemaphoreType.DMA,
    pltpu.SemaphoreType.DMA,
    pltpu.SemaphoreType.DMA,
    pltpu.SemaphoreType.DMA,
]

_NACC = 4


def _sc_gather_sum_body(rs_hbm, it0_hbm, it1_hbm, out_hbm,
                        lut0, lut1, idx0_v, idx1_v, acc_v,
                        sem0, sem1, sem2, sem3):
    wid = lax.axis_index("s") * NC + lax.axis_index("c")
    base = wid * COLS_W
    cp_l0 = pltpu.async_copy(rs_hbm.at[0], lut0, sem0)
    cp_i0 = pltpu.async_copy(it0_hbm.at[:, pl.ds(base, COLS_W)], idx0_v, sem1)
    cp_l1 = pltpu.async_copy(rs_hbm.at[1], lut1, sem2)
    cp_i1 = pltpu.async_copy(it1_hbm.at[:, pl.ds(base, COLS_W)], idx1_v, sem3)

    npc = COLS_W // L
    zeros = tuple(jnp.zeros((L,), jnp.float32) for _ in range(_NACC))

    def table_loop(lut, idx_v, accs):
        def row_body(r, accs):
            accs = list(accs)
            for c in range(npc):
                iv = idx_v[r, pl.ds(c * L, L)]
                accs[c % _NACC] = accs[c % _NACC] + plsc.load_gather(lut, [iv])
            return tuple(accs)
        return lax.fori_loop(0, POOL, row_body, accs)

    cp_l0.wait()
    cp_i0.wait()
    accs = table_loop(lut0, idx0_v, zeros)
    cp_l1.wait()
    cp_i1.wait()
    accs = table_loop(lut1, idx1_v, accs)
    acc_v[...] = (accs[0] + accs[1]) + (accs[2] + accs[3])
    pltpu.sync_copy(acc_v, out_hbm.at[wid])


_sc_gather_sum = pl.kernel(
    _sc_gather_sum_body,
    out_type=jax.ShapeDtypeStruct((NW, L), jnp.float32),
    mesh=_mesh,
    scratch_types=_SC_SCRATCH,
    compiler_params=pltpu.CompilerParams(needs_layout_passes=False),
)


def kernel(indices_0, indices_1, table_0, table_1):
    rs = _rowsum(table_0.T, table_1.T)
    partials = _sc_gather_sum(rs, indices_0.T, indices_1.T)
    return jnp.sum(partials) / jnp.float32(BATCH * 2 * EMBED_DIM)
